# Initial kernel scaffold; baseline (speedup 1.0000x reference)
import functools

import jax
import jax.numpy as jnp
from jax import lax
from jax.experimental import pallas as pl
from jax.experimental.pallas import tpu as pltpu
from jax.experimental.pallas import tpu_sc as plsc

_N = 100000
_E = 1600000
_NPAD = 100096
_RPT = _NPAD // 16
_R = 12544
_EP = _R * 128
_G = 56

_mesh = plsc.VectorSubcoreMesh(core_axis_name="c", subcore_axis_name="s")


@functools.partial(
    pl.kernel,
    mesh=_mesh,
    out_type=jax.ShapeDtypeStruct((2, _NPAD), jnp.float32),
    scratch_types=[
        pltpu.VMEM((_G, 128), jnp.int32),
        pltpu.VMEM((128,), jnp.float32),
        pltpu.VMEM_SHARED((_NPAD,), jnp.float32),
    ],
)
def _sc_degree(dst_hbm, zrow_hbm, out_hbm, dst_v, ones_v, acc):
    c = lax.axis_index("c")
    s = lax.axis_index("s")
    pltpu.sync_copy(zrow_hbm, acc.at[pl.ds(s * _RPT, _RPT)])
    for i in range(8):
        ones_v[pl.ds(16 * i, 16)] = jnp.ones((16,), jnp.float32)
    plsc.subcore_barrier()
    base = c * (_R // 2) + s * 392
    def group(g, carry):
        row0 = base + g * _G
        pltpu.sync_copy(dst_hbm.at[pl.ds(row0, _G)], dst_v)
        def step(j, carry2):
            pltpu.sync_copy(ones_v, acc.at[dst_v.at[j]], add=True)
            return carry2
        return lax.fori_loop(0, _G, step, carry)
    lax.fori_loop(0, 7, group, 0)
    plsc.subcore_barrier()
    pltpu.sync_copy(acc.at[pl.ds(s * _RPT, _RPT)],
                    out_hbm.at[c, pl.ds(s * _RPT, _RPT)])


def _make_sc_agg(F, per_core_src, rows_per_tile, n_groups):
    @functools.partial(
        pl.kernel,
        mesh=_mesh,
        out_type=jax.ShapeDtypeStruct((2, _NPAD, F), jnp.float32),
        scratch_types=[
            pltpu.VMEM((_G, 128), jnp.int32),
            pltpu.VMEM((_G, 128), jnp.int32),
            pltpu.VMEM((128, F), jnp.float32),
            pltpu.VMEM_SHARED((_NPAD, F), jnp.float32),
            pltpu.SemaphoreType.DMA,
        ],
    )
    def agg(src_hbm, dst_hbm, u_hbm, z_hbm, out_hbm, src_v, dst_v, rows_v, acc, sem):
        c = lax.axis_index("c")
        s = lax.axis_index("s")
        pltpu.sync_copy(z_hbm, acc.at[pl.ds(s * _RPT, _RPT)])
        plsc.subcore_barrier()
        if per_core_src:
            base = s * rows_per_tile
        else:
            base = c * (_R // 2) + s * rows_per_tile
        def group(g, carry):
            row0 = base + g * _G
            if per_core_src:
                pltpu.sync_copy(src_hbm.at[c, pl.ds(row0, _G)], src_v)
            else:
                pltpu.sync_copy(src_hbm.at[pl.ds(row0, _G)], src_v)
            pltpu.sync_copy(dst_hbm.at[pl.ds(row0, _G)], dst_v)
            def step(j, carry2):
                pltpu.async_copy(u_hbm.at[src_v.at[j]], rows_v, sem).wait()
                pltpu.sync_copy(rows_v, acc.at[dst_v.at[j]], add=True)
                return carry2
            return lax.fori_loop(0, _G, step, carry)
        lax.fori_loop(0, n_groups, group, 0)
        plsc.subcore_barrier()
        pltpu.sync_copy(acc.at[pl.ds(s * _RPT, _RPT)],
                        out_hbm.at[c, pl.ds(s * _RPT, _RPT)])
    return agg


_sc_agg8 = _make_sc_agg(8, False, 392, 7)
_sc_agg16 = _make_sc_agg(16, True, 784, 14)


_BR = 1000
_GRID = _N // _BR


def _tc_pre_body(d0, d1, x, dinv_o, u1_o):
    deg = d0[...] + d1[...] + 1.0
    dv = lax.rsqrt(deg)
    dinv_o[...] = dv
    u = dv * x[...]
    u1_o[...] = jnp.concatenate([u, jnp.zeros((_BR, 5), jnp.float32)], axis=1)


def _tc_pre(d0, d1, x):
    return pl.pallas_call(
        _tc_pre_body,
        grid=(_GRID,),
        in_specs=[
            pl.BlockSpec((_BR, 1), lambda i: (i, 0)),
            pl.BlockSpec((_BR, 1), lambda i: (i, 0)),
            pl.BlockSpec((_BR, 3), lambda i: (i, 0)),
        ],
        out_specs=[
            pl.BlockSpec((_BR, 1), lambda i: (i, 0)),
            pl.BlockSpec((_BR, 8), lambda i: (i, 0)),
        ],
        out_shape=[
            jax.ShapeDtypeStruct((_N, 1), jnp.float32),
            jax.ShapeDtypeStruct((_N, 8), jnp.float32),
        ],
    )(d0, d1, x)


def _tc_mid_body(a0, a1, u1, dinv, W1, b1, W2, u2_o):
    s1 = a0[...] + a1[...] + u1[...]
    sn = dinv[...] * s1
    h1 = jnp.maximum(
        jnp.dot(sn[:, :3], W1[...], preferred_element_type=jnp.float32) + b1[...],
        0.0)
    t = jnp.dot(h1, W2[...], preferred_element_type=jnp.float32)
    u2 = dinv[...] * t
    u2_o[0] = u2[:, :16]
    u2_o[1] = u2[:, 16:]


def _tc_mid(a0, a1, u1, dinv, W1, b1, W2):
    return pl.pallas_call(
        _tc_mid_body,
        grid=(_GRID,),
        in_specs=[
            pl.BlockSpec((_BR, 8), lambda i: (i, 0)),
            pl.BlockSpec((_BR, 8), lambda i: (i, 0)),
            pl.BlockSpec((_BR, 8), lambda i: (i, 0)),
            pl.BlockSpec((_BR, 1), lambda i: (i, 0)),
            pl.BlockSpec((3, 64), lambda i: (0, 0)),
            pl.BlockSpec((64,), lambda i: (0,)),
            pl.BlockSpec((64, 32), lambda i: (0, 0)),
        ],
        out_specs=pl.BlockSpec((2, _BR, 16), lambda i: (0, i, 0)),
        out_shape=jax.ShapeDtypeStruct((2, _N, 16), jnp.float32),
    )(a0, a1, u1, dinv, W1, b1, W2)


def _tc_head_body(a2, u2, dinv, b2, Wh1, bh1, Wh2, bh2, p_o):
    s2 = a2[...] + u2[...]
    emb = jnp.concatenate([s2[0], s2[1]], axis=1)
    emb = dinv[...] * emb + b2[...]
    z = jnp.maximum(
        jnp.dot(emb, Wh1[...], preferred_element_type=jnp.float32) + bh1[...],
        0.0)
    p_o[...] = jnp.dot(z, Wh2[...], preferred_element_type=jnp.float32) + bh2[...]


def _tc_head(a2, u2, dinv, b2, Wh1, bh1, Wh2, bh2):
    return pl.pallas_call(
        _tc_head_body,
        grid=(_GRID,),
        in_specs=[
            pl.BlockSpec((2, _BR, 16), lambda i: (0, i, 0)),
            pl.BlockSpec((2, _BR, 16), lambda i: (0, i, 0)),
            pl.BlockSpec((_BR, 1), lambda i: (i, 0)),
            pl.BlockSpec((32,), lambda i: (0,)),
            pl.BlockSpec((32, 32), lambda i: (0, 0)),
            pl.BlockSpec((32,), lambda i: (0,)),
            pl.BlockSpec((32, 1), lambda i: (0, 0)),
            pl.BlockSpec((1,), lambda i: (0,)),
        ],
        out_specs=pl.BlockSpec((_BR, 1), lambda i: (i, 0)),
        out_shape=jax.ShapeDtypeStruct((_N, 1), jnp.float32),
    )(a2, u2, dinv, b2, Wh1, bh1, Wh2, bh2)


def kernel(x, edge_index, W1, b1, W2, b2, Wh1, bh1, Wh2, bh2):
    ei = edge_index.astype(jnp.int32)
    padn = _EP - _E
    src_p = jnp.concatenate([ei[0], jnp.zeros((padn,), jnp.int32)])
    dst_p = jnp.concatenate([ei[1], jnp.full((padn,), _N, jnp.int32)])
    src_r = src_p.reshape(_R, 128)
    dst_r = dst_p.reshape(_R, 128)
    src2_r = jnp.stack([src_r, src_r + _N])
    z1 = jnp.zeros((_RPT,), jnp.float32)
    z8 = jnp.zeros((_RPT, 8), jnp.float32)
    z16 = jnp.zeros((_RPT, 16), jnp.float32)

    degs = _sc_degree(dst_r, z1)
    d0 = degs[0, :_N].reshape(_N, 1)
    d1 = degs[1, :_N].reshape(_N, 1)
    dinv, u1 = _tc_pre(d0, d1, x)

    acc1 = _sc_agg8(src_r, dst_r, u1, z8)
    u2 = _tc_mid(acc1[0, :_N], acc1[1, :_N], u1, dinv, W1, b1, W2)

    acc2 = _sc_agg16(src2_r, dst_r, u2.reshape(2 * _N, 16), z16)
    p = _tc_head(acc2[:, :_N, :], u2, dinv, b2, Wh1, bh1, Wh2, bh2)
    return p.reshape(_N)


# speedup vs baseline: 14.5341x; 14.5341x over previous
import functools

import jax
import jax.numpy as jnp
from jax import lax
from jax.experimental import pallas as pl
from jax.experimental.pallas import tpu as pltpu
from jax.experimental.pallas import tpu_sc as plsc

_N = 100000
_E = 1600000
_NPAD = 100096
_RPT = _NPAD // 16
_HRPT = _RPT // 2
_R = 12544
_EP = _R * 128
_G = 56
_F = 8

_mesh = plsc.VectorSubcoreMesh(core_axis_name="c", subcore_axis_name="s")
_params = pltpu.CompilerParams(use_tc_tiling_on_sc=False)


@functools.partial(
    pl.kernel,
    mesh=_mesh,
    compiler_params=_params,
    out_type=jax.ShapeDtypeStruct((2 * _NPAD,), jnp.float32),
    scratch_types=[
        pltpu.VMEM((_G, 128), jnp.int32),
        pltpu.VMEM((128,), jnp.float32),
        pltpu.VMEM((_HRPT,), jnp.float32),
        pltpu.VMEM_SHARED((_NPAD,), jnp.float32),
    ],
)
def _sc_degree(dst_hbm, zrow_hbm, out_hbm, dst_v, ones_v, vbuf, acc):
    c = lax.axis_index("c")
    s = lax.axis_index("s")
    pltpu.sync_copy(zrow_hbm, vbuf)
    for k in range(2):
        pltpu.sync_copy(vbuf, acc.at[pl.ds(s * _RPT + k * _HRPT, _HRPT)])
    for i in range(8):
        ones_v[pl.ds(16 * i, 16)] = jnp.ones((16,), jnp.float32)
    plsc.subcore_barrier()
    base = c * (_R // 2) + s * 392
    def group(g, carry):
        row0 = base + g * _G
        pltpu.sync_copy(dst_hbm.at[pl.ds(row0, _G)], dst_v)
        def step(j, carry2):
            pltpu.sync_copy(ones_v, acc.at[dst_v.at[j]], add=True)
            return carry2
        return lax.fori_loop(0, _G, step, carry)
    lax.fori_loop(0, 7, group, 0)
    plsc.subcore_barrier()
    for k in range(2):
        off = s * _RPT + k * _HRPT
        pltpu.sync_copy(acc.at[pl.ds(off, _HRPT)], vbuf)
        pltpu.sync_copy(vbuf, out_hbm.at[pl.ds(c * _NPAD + off, _HRPT)])


def _zero_acc(zrow_hbm, vbuf, acc, s):
    pltpu.sync_copy(zrow_hbm, vbuf)
    for k in range(2):
        pltpu.sync_copy(vbuf, acc.at[pl.ds(s * _RPT + k * _HRPT, _HRPT)])


def _drain_acc(acc, vbuf, out_hbm, oidx, s):
    for k in range(2):
        off = s * _RPT + k * _HRPT
        pltpu.sync_copy(acc.at[pl.ds(off, _HRPT)], vbuf)
        pltpu.sync_copy(vbuf, out_hbm.at[oidx, pl.ds(off, _HRPT)])


@functools.partial(
    pl.kernel,
    mesh=_mesh,
    compiler_params=_params,
    out_type=jax.ShapeDtypeStruct((2, _NPAD, _F), jnp.float32),
    scratch_types=[
        pltpu.VMEM((_G, 128), jnp.int32),
        pltpu.VMEM((_G, 128), jnp.int32),
        pltpu.VMEM((128, _F), jnp.float32),
        pltpu.VMEM((_HRPT, _F), jnp.float32),
        pltpu.VMEM_SHARED((_NPAD, _F), jnp.float32),
        pltpu.SemaphoreType.DMA,
    ],
)
def _sc_agg1(src_hbm, dst_hbm, u_hbm, z_hbm, out_hbm,
             src_v, dst_v, rows_v, vbuf, acc, sem):
    c = lax.axis_index("c")
    s = lax.axis_index("s")
    _zero_acc(z_hbm, vbuf, acc, s)
    plsc.subcore_barrier()
    base = c * (_R // 2) + s * 392
    def group(g, carry):
        row0 = base + g * _G
        pltpu.sync_copy(src_hbm.at[pl.ds(row0, _G)], src_v)
        pltpu.sync_copy(dst_hbm.at[pl.ds(row0, _G)], dst_v)
        def step(j, carry2):
            pltpu.async_copy(u_hbm.at[src_v.at[j]], rows_v, sem).wait()
            pltpu.sync_copy(rows_v, acc.at[dst_v.at[j]], add=True)
            return carry2
        return lax.fori_loop(0, _G, step, carry)
    lax.fori_loop(0, 7, group, 0)
    plsc.subcore_barrier()
    _drain_acc(acc, vbuf, out_hbm, c, s)


@functools.partial(
    pl.kernel,
    mesh=_mesh,
    compiler_params=_params,
    out_type=jax.ShapeDtypeStruct((4, _NPAD, _F), jnp.float32),
    scratch_types=[
        pltpu.VMEM((_G, 128), jnp.int32),
        pltpu.VMEM((_G, 128), jnp.int32),
        pltpu.VMEM((128, _F), jnp.float32),
        pltpu.VMEM((_HRPT, _F), jnp.float32),
        pltpu.VMEM_SHARED((_NPAD, _F), jnp.float32),
        pltpu.SemaphoreType.DMA,
    ],
)
def _sc_agg2(src4_hbm, dst_hbm, u_hbm, z_hbm, out_hbm,
             src_v, dst_v, rows_v, vbuf, acc, sem):
    c = lax.axis_index("c")
    s = lax.axis_index("s")
    for p in range(2):
        _zero_acc(z_hbm, vbuf, acc, s)
        plsc.subcore_barrier()
        gidx = 2 * c + p
        base = s * 784
        def group(g, carry):
            row0 = base + g * _G
            pltpu.sync_copy(src4_hbm.at[gidx, pl.ds(row0, _G)], src_v)
            pltpu.sync_copy(dst_hbm.at[pl.ds(row0, _G)], dst_v)
            def step(j, carry2):
                pltpu.async_copy(u_hbm.at[src_v.at[j]], rows_v, sem).wait()
                pltpu.sync_copy(rows_v, acc.at[dst_v.at[j]], add=True)
                return carry2
            return lax.fori_loop(0, _G, step, carry)
        lax.fori_loop(0, 14, group, 0)
        plsc.subcore_barrier()
        _drain_acc(acc, vbuf, out_hbm, gidx, s)
        plsc.subcore_barrier()


_BR = 1000
_GRID = _N // _BR


def _tc_pre_body(d0, d1, x, dinv_o, u1_o):
    deg = d0[...] + d1[...] + 1.0
    dv = lax.rsqrt(deg)
    dinv_o[...] = dv
    u = dv * x[...]
    u1_o[...] = jnp.concatenate([u, jnp.zeros((_BR, 5), jnp.float32)], axis=1)


def _tc_pre(d0, d1, x):
    return pl.pallas_call(
        _tc_pre_body,
        grid=(_GRID,),
        in_specs=[
            pl.BlockSpec((_BR, 1), lambda i: (i, 0)),
            pl.BlockSpec((_BR, 1), lambda i: (i, 0)),
            pl.BlockSpec((_BR, 3), lambda i: (i, 0)),
        ],
        out_specs=[
            pl.BlockSpec((_BR, 1), lambda i: (i, 0)),
            pl.BlockSpec((_BR, 8), lambda i: (i, 0)),
        ],
        out_shape=[
            jax.ShapeDtypeStruct((_N, 1), jnp.float32),
            jax.ShapeDtypeStruct((_N, 8), jnp.float32),
        ],
    )(d0, d1, x)


def _tc_mid_body(a0, a1, u1, dinv, W1, b1, W2, u2_o):
    s1 = a0[...] + a1[...] + u1[...]
    sn = dinv[...] * s1
    h1 = jnp.maximum(
        jnp.dot(sn[:, :3], W1[...], preferred_element_type=jnp.float32) + b1[...],
        0.0)
    t = jnp.dot(h1, W2[...], preferred_element_type=jnp.float32)
    u2 = dinv[...] * t
    for g in range(4):
        u2_o[g] = u2[:, 8 * g:8 * g + 8]


def _tc_mid(a0, a1, u1, dinv, W1, b1, W2):
    return pl.pallas_call(
        _tc_mid_body,
        grid=(_GRID,),
        in_specs=[
            pl.BlockSpec((_BR, 8), lambda i: (i, 0)),
            pl.BlockSpec((_BR, 8), lambda i: (i, 0)),
            pl.BlockSpec((_BR, 8), lambda i: (i, 0)),
            pl.BlockSpec((_BR, 1), lambda i: (i, 0)),
            pl.BlockSpec((3, 64), lambda i: (0, 0)),
            pl.BlockSpec((64,), lambda i: (0,)),
            pl.BlockSpec((64, 32), lambda i: (0, 0)),
        ],
        out_specs=pl.BlockSpec((4, _BR, 8), lambda i: (0, i, 0)),
        out_shape=jax.ShapeDtypeStruct((4, _N, 8), jnp.float32),
    )(a0, a1, u1, dinv, W1, b1, W2)


def _tc_head_body(a2, u2, dinv, b2, Wh1, bh1, Wh2, bh2, p_o):
    s2 = a2[...] + u2[...]
    emb = jnp.concatenate([s2[g] for g in range(4)], axis=1)
    emb = dinv[...] * emb + b2[...]
    z = jnp.maximum(
        jnp.dot(emb, Wh1[...], preferred_element_type=jnp.float32) + bh1[...],
        0.0)
    p_o[...] = jnp.dot(z, Wh2[...], preferred_element_type=jnp.float32) + bh2[...]


def _tc_head(a2, u2, dinv, b2, Wh1, bh1, Wh2, bh2):
    return pl.pallas_call(
        _tc_head_body,
        grid=(_GRID,),
        in_specs=[
            pl.BlockSpec((4, _BR, 8), lambda i: (0, i, 0)),
            pl.BlockSpec((4, _BR, 8), lambda i: (0, i, 0)),
            pl.BlockSpec((_BR, 1), lambda i: (i, 0)),
            pl.BlockSpec((32,), lambda i: (0,)),
            pl.BlockSpec((32, 32), lambda i: (0, 0)),
            pl.BlockSpec((32,), lambda i: (0,)),
            pl.BlockSpec((32, 1), lambda i: (0, 0)),
            pl.BlockSpec((1,), lambda i: (0,)),
        ],
        out_specs=pl.BlockSpec((_BR, 1), lambda i: (i, 0)),
        out_shape=jax.ShapeDtypeStruct((_N, 1), jnp.float32),
    )(a2, u2, dinv, b2, Wh1, bh1, Wh2, bh2)


def kernel(x, edge_index, W1, b1, W2, b2, Wh1, bh1, Wh2, bh2):
    ei = edge_index.astype(jnp.int32)
    padn = _EP - _E
    src_p = jnp.concatenate([ei[0], jnp.zeros((padn,), jnp.int32)])
    dst_p = jnp.concatenate([ei[1], jnp.full((padn,), _N, jnp.int32)])
    src_r = src_p.reshape(_R, 128)
    dst_r = dst_p.reshape(_R, 128)
    src4_r = jnp.stack([src_r + g * _N for g in range(4)])
    z1 = jnp.zeros((_HRPT,), jnp.float32)
    z8 = jnp.zeros((_HRPT, _F), jnp.float32)

    degs = _sc_degree(dst_r, z1).reshape(2, _NPAD)
    d0 = degs[0, :_N].reshape(_N, 1)
    d1 = degs[1, :_N].reshape(_N, 1)
    dinv, u1 = _tc_pre(d0, d1, x)

    acc1 = _sc_agg1(src_r, dst_r, u1, z8)
    u2 = _tc_mid(acc1[0, :_N], acc1[1, :_N], u1, dinv, W1, b1, W2)

    acc2 = _sc_agg2(src4_r, dst_r, u2.reshape(4 * _N, _F), z8)
    p = _tc_head(acc2[:, :_N, :], u2, dinv, b2, Wh1, bh1, Wh2, bh2)
    return p.reshape(_N)


# speedup vs baseline: 16.9893x; 1.1689x over previous
import functools

import jax
import jax.numpy as jnp
from jax import lax
from jax.experimental import pallas as pl
from jax.experimental.pallas import tpu as pltpu
from jax.experimental.pallas import tpu_sc as plsc

_N = 100000
_E = 1600000
_NPAD = 100096
_RPT = _NPAD // 16
_HRPT = _RPT // 2
_R = 12544
_EP = _R * 128
_G = 56
_F = 8

_mesh = plsc.VectorSubcoreMesh(core_axis_name="c", subcore_axis_name="s")
_params = pltpu.CompilerParams(use_tc_tiling_on_sc=False)


@functools.partial(
    pl.kernel,
    mesh=_mesh,
    compiler_params=_params,
    out_type=jax.ShapeDtypeStruct((2 * _NPAD,), jnp.float32),
    scratch_types=[
        pltpu.VMEM((_G, 128), jnp.int32),
        pltpu.VMEM((128,), jnp.float32),
        pltpu.VMEM((_HRPT,), jnp.float32),
        pltpu.VMEM_SHARED((_NPAD,), jnp.float32),
    ],
)
def _sc_degree(dst_hbm, zrow_hbm, out_hbm, dst_v, ones_v, vbuf, acc):
    c = lax.axis_index("c")
    s = lax.axis_index("s")
    pltpu.sync_copy(zrow_hbm, vbuf)
    for k in range(2):
        pltpu.sync_copy(vbuf, acc.at[pl.ds(s * _RPT + k * _HRPT, _HRPT)])
    for i in range(8):
        ones_v[pl.ds(16 * i, 16)] = jnp.ones((16,), jnp.float32)
    plsc.subcore_barrier()
    base = c * (_R // 2) + s * 392
    def group(g, carry):
        row0 = base + g * _G
        pltpu.sync_copy(dst_hbm.at[pl.ds(row0, _G)], dst_v)
        def step(j, carry2):
            pltpu.sync_copy(ones_v, acc.at[dst_v.at[j]], add=True)
            return carry2
        return lax.fori_loop(0, _G, step, carry)
    lax.fori_loop(0, 7, group, 0)
    plsc.subcore_barrier()
    for k in range(2):
        off = s * _RPT + k * _HRPT
        pltpu.sync_copy(acc.at[pl.ds(off, _HRPT)], vbuf)
        pltpu.sync_copy(vbuf, out_hbm.at[pl.ds(c * _NPAD + off, _HRPT)])


def _zero_acc(zrow_hbm, vbuf, acc, s):
    pltpu.sync_copy(zrow_hbm, vbuf)
    for k in range(2):
        pltpu.sync_copy(vbuf, acc.at[pl.ds(s * _RPT + k * _HRPT, _HRPT)])


def _drain_acc(acc, vbuf, out_hbm, oidx, s):
    for k in range(2):
        off = s * _RPT + k * _HRPT
        pltpu.sync_copy(acc.at[pl.ds(off, _HRPT)], vbuf)
        pltpu.sync_copy(vbuf, out_hbm.at[oidx, pl.ds(off, _HRPT)])


def _agg_group(u_hbm, acc, src_v, dst_v, rows_a, rows_b, sem_a, sem_b):
    pltpu.async_copy(u_hbm.at[src_v.at[0]], rows_a, sem_a)
    def pair(i, carry):
        j0 = 2 * i
        pltpu.async_copy(u_hbm.at[src_v.at[j0 + 1]], rows_b, sem_b)
        pltpu.make_async_copy(u_hbm.at[src_v.at[j0]], rows_a, sem_a).wait()
        pltpu.sync_copy(rows_a, acc.at[dst_v.at[j0]], add=True)
        jn = jnp.minimum(j0 + 2, _G - 1)
        pltpu.async_copy(u_hbm.at[src_v.at[jn]], rows_a, sem_a)
        pltpu.make_async_copy(u_hbm.at[src_v.at[j0 + 1]], rows_b, sem_b).wait()
        pltpu.sync_copy(rows_b, acc.at[dst_v.at[j0 + 1]], add=True)
        return carry
    lax.fori_loop(0, _G // 2, pair, 0)
    pltpu.make_async_copy(u_hbm.at[src_v.at[_G - 1]], rows_a, sem_a).wait()


def _make_sc_agg(n_out, n_passes, n_groups, src3d):
    @functools.partial(
        pl.kernel,
        mesh=_mesh,
        compiler_params=_params,
        out_type=jax.ShapeDtypeStruct((n_out, _NPAD, _F), jnp.float32),
        scratch_types=[
            pltpu.VMEM((_G, 128), jnp.int32),
            pltpu.VMEM((_G, 128), jnp.int32),
            pltpu.VMEM((128, _F), jnp.float32),
            pltpu.VMEM((128, _F), jnp.float32),
            pltpu.VMEM((_HRPT, _F), jnp.float32),
            pltpu.VMEM_SHARED((_NPAD, _F), jnp.float32),
            pltpu.SemaphoreType.DMA,
            pltpu.SemaphoreType.DMA,
        ],
    )
    def agg(src_hbm, dst_hbm, u_hbm, z_hbm, out_hbm,
            src_v, dst_v, rows_a, rows_b, vbuf, acc, sem_a, sem_b):
        c = lax.axis_index("c")
        s = lax.axis_index("s")
        for p in range(n_passes):
            _zero_acc(z_hbm, vbuf, acc, s)
            plsc.subcore_barrier()
            if src3d:
                gidx = 2 * c + p
                base = s * (n_groups * _G)
            else:
                gidx = c
                base = c * (_R // 2) + s * (n_groups * _G)
            def group(g, carry):
                row0 = base + g * _G
                if src3d:
                    pltpu.sync_copy(src_hbm.at[gidx, pl.ds(row0, _G)], src_v)
                else:
                    pltpu.sync_copy(src_hbm.at[pl.ds(row0, _G)], src_v)
                pltpu.sync_copy(dst_hbm.at[pl.ds(row0, _G)], dst_v)
                _agg_group(u_hbm, acc, src_v, dst_v, rows_a, rows_b, sem_a, sem_b)
                return carry
            lax.fori_loop(0, n_groups, group, 0)
            plsc.subcore_barrier()
            _drain_acc(acc, vbuf, out_hbm, gidx, s)
            if p + 1 < n_passes:
                plsc.subcore_barrier()
    return agg


_sc_agg1 = _make_sc_agg(2, 1, 7, False)
_sc_agg2 = _make_sc_agg(4, 2, 14, True)


_BR = 1000
_GRID = _N // _BR


def _tc_pre_body(d0, d1, x, dinv_o, u1_o):
    deg = d0[...] + d1[...] + 1.0
    dv = lax.rsqrt(deg)
    dinv_o[...] = dv
    u = dv * x[...]
    u1_o[...] = jnp.concatenate([u, jnp.zeros((_BR, 5), jnp.float32)], axis=1)


def _tc_pre(d0, d1, x):
    return pl.pallas_call(
        _tc_pre_body,
        grid=(_GRID,),
        in_specs=[
            pl.BlockSpec((_BR, 1), lambda i: (i, 0)),
            pl.BlockSpec((_BR, 1), lambda i: (i, 0)),
            pl.BlockSpec((_BR, 3), lambda i: (i, 0)),
        ],
        out_specs=[
            pl.BlockSpec((_BR, 1), lambda i: (i, 0)),
            pl.BlockSpec((_BR, 8), lambda i: (i, 0)),
        ],
        out_shape=[
            jax.ShapeDtypeStruct((_N, 1), jnp.float32),
            jax.ShapeDtypeStruct((_N, 8), jnp.float32),
        ],
    )(d0, d1, x)


def _tc_mid_body(a0, a1, u1, dinv, W1, b1, W2, u2_o):
    s1 = a0[...] + a1[...] + u1[...]
    sn = dinv[...] * s1
    h1 = jnp.maximum(
        jnp.dot(sn[:, :3], W1[...], preferred_element_type=jnp.float32, precision=lax.Precision.HIGHEST) + b1[...],
        0.0)
    t = jnp.dot(h1, W2[...], preferred_element_type=jnp.float32, precision=lax.Precision.HIGHEST)
    u2 = dinv[...] * t
    for g in range(4):
        u2_o[g] = u2[:, 8 * g:8 * g + 8]


def _tc_mid(a0, a1, u1, dinv, W1, b1, W2):
    return pl.pallas_call(
        _tc_mid_body,
        grid=(_GRID,),
        in_specs=[
            pl.BlockSpec((_BR, 8), lambda i: (i, 0)),
            pl.BlockSpec((_BR, 8), lambda i: (i, 0)),
            pl.BlockSpec((_BR, 8), lambda i: (i, 0)),
            pl.BlockSpec((_BR, 1), lambda i: (i, 0)),
            pl.BlockSpec((3, 64), lambda i: (0, 0)),
            pl.BlockSpec((64,), lambda i: (0,)),
            pl.BlockSpec((64, 32), lambda i: (0, 0)),
        ],
        out_specs=pl.BlockSpec((4, _BR, 8), lambda i: (0, i, 0)),
        out_shape=jax.ShapeDtypeStruct((4, _N, 8), jnp.float32),
    )(a0, a1, u1, dinv, W1, b1, W2)


def _tc_head_body(a2, u2, dinv, b2, Wh1, bh1, Wh2, bh2, p_o):
    s2 = a2[...] + u2[...]
    emb = jnp.concatenate([s2[g] for g in range(4)], axis=1)
    emb = dinv[...] * emb + b2[...]
    z = jnp.maximum(
        jnp.dot(emb, Wh1[...], preferred_element_type=jnp.float32, precision=lax.Precision.HIGHEST) + bh1[...],
        0.0)
    p_o[...] = jnp.dot(z, Wh2[...], preferred_element_type=jnp.float32, precision=lax.Precision.HIGHEST) + bh2[...]


def _tc_head(a2, u2, dinv, b2, Wh1, bh1, Wh2, bh2):
    return pl.pallas_call(
        _tc_head_body,
        grid=(_GRID,),
        in_specs=[
            pl.BlockSpec((4, _BR, 8), lambda i: (0, i, 0)),
            pl.BlockSpec((4, _BR, 8), lambda i: (0, i, 0)),
            pl.BlockSpec((_BR, 1), lambda i: (i, 0)),
            pl.BlockSpec((32,), lambda i: (0,)),
            pl.BlockSpec((32, 32), lambda i: (0, 0)),
            pl.BlockSpec((32,), lambda i: (0,)),
            pl.BlockSpec((32, 1), lambda i: (0, 0)),
            pl.BlockSpec((1,), lambda i: (0,)),
        ],
        out_specs=pl.BlockSpec((_BR, 1), lambda i: (i, 0)),
        out_shape=jax.ShapeDtypeStruct((_N, 1), jnp.float32),
    )(a2, u2, dinv, b2, Wh1, bh1, Wh2, bh2)


def kernel(x, edge_index, W1, b1, W2, b2, Wh1, bh1, Wh2, bh2):
    ei = edge_index.astype(jnp.int32)
    padn = _EP - _E
    src_p = jnp.concatenate([ei[0], jnp.zeros((padn,), jnp.int32)])
    dst_p = jnp.concatenate([ei[1], jnp.full((padn,), _N, jnp.int32)])
    src_r = src_p.reshape(_R, 128)
    dst_r = dst_p.reshape(_R, 128)
    src4_r = jnp.stack([src_r + g * _N for g in range(4)])
    z1 = jnp.zeros((_HRPT,), jnp.float32)
    z8 = jnp.zeros((_HRPT, _F), jnp.float32)

    degs = _sc_degree(dst_r, z1).reshape(2, _NPAD)
    d0 = degs[0, :_N].reshape(_N, 1)
    d1 = degs[1, :_N].reshape(_N, 1)
    dinv, u1 = _tc_pre(d0, d1, x)

    acc1 = _sc_agg1(src_r, dst_r, u1, z8)
    u2 = _tc_mid(acc1[0, :_N], acc1[1, :_N], u1, dinv, W1, b1, W2)

    acc2 = _sc_agg2(src4_r, dst_r, u2.reshape(4 * _N, _F), z8)
    p = _tc_head(acc2[:, :_N, :], u2, dinv, b2, Wh1, bh1, Wh2, bh2)
    return p.reshape(_N)


# speedup vs baseline: 22.2240x; 1.3081x over previous
import functools

import jax
import jax.numpy as jnp
from jax import lax
from jax.experimental import pallas as pl
from jax.experimental.pallas import tpu as pltpu
from jax.experimental.pallas import tpu_sc as plsc

_N = 100000
_E = 1600000
_NPAD = 100096
_RPT = _NPAD // 16
_HRPT = _RPT // 2
_R = 12544
_EP = _R * 128
_G = 56
_F = 8
_CHK = 391
_NCHK = 16

_mesh = plsc.VectorSubcoreMesh(core_axis_name="c", subcore_axis_name="s")
_params = pltpu.CompilerParams(use_tc_tiling_on_sc=False)


@functools.partial(
    pl.kernel,
    mesh=_mesh,
    compiler_params=_params,
    out_type=jax.ShapeDtypeStruct((2 * _NPAD,), jnp.float32),
    scratch_types=[
        pltpu.VMEM((_G, 128), jnp.int32),
        pltpu.VMEM((128,), jnp.float32),
        pltpu.VMEM((_HRPT,), jnp.float32),
        pltpu.VMEM_SHARED((_NPAD,), jnp.float32),
    ],
)
def _sc_degree(dst_hbm, zrow_hbm, out_hbm, dst_v, ones_v, vbuf, acc):
    c = lax.axis_index("c")
    s = lax.axis_index("s")
    pltpu.sync_copy(zrow_hbm, vbuf)
    for k in range(2):
        pltpu.sync_copy(vbuf, acc.at[pl.ds(s * _RPT + k * _HRPT, _HRPT)])
    for i in range(8):
        ones_v[pl.ds(16 * i, 16)] = jnp.ones((16,), jnp.float32)
    plsc.subcore_barrier()
    base = c * (_R // 2) + s * 392
    def group(g, carry):
        row0 = base + g * _G
        pltpu.sync_copy(dst_hbm.at[pl.ds(row0, _G)], dst_v)
        def step(j, carry2):
            pltpu.sync_copy(ones_v, acc.at[dst_v.at[j]], add=True)
            return carry2
        return lax.fori_loop(0, _G, step, carry)
    lax.fori_loop(0, 7, group, 0)
    plsc.subcore_barrier()
    for k in range(2):
        off = s * _RPT + k * _HRPT
        pltpu.sync_copy(acc.at[pl.ds(off, _HRPT)], vbuf)
        pltpu.sync_copy(vbuf, out_hbm.at[pl.ds(c * _NPAD + off, _HRPT)])


def _zero_acc(zrow_hbm, vbuf, acc, s):
    pltpu.sync_copy(zrow_hbm, vbuf)
    def zstep(k, carry):
        pltpu.sync_copy(vbuf, acc.at[pl.ds(s * _RPT + k * _CHK, _CHK)])
        return carry
    lax.fori_loop(0, _NCHK, zstep, 0)


def _drain_acc(acc, vbuf, out_hbm, oidx, s):
    def dstep(k, carry):
        off = s * _RPT + k * _CHK
        pltpu.sync_copy(acc.at[pl.ds(off, _CHK)], vbuf)
        pltpu.sync_copy(vbuf, out_hbm.at[oidx, pl.ds(off, _CHK)])
        return carry
    lax.fori_loop(0, _NCHK, dstep, 0)


def _agg_group(u_hbm, acc, src_v, dst_v, rows_a, rows_b, sem_a, sem_b):
    pltpu.async_copy(u_hbm.at[src_v.at[0]], rows_a, sem_a)
    def pair(i, carry):
        j0 = 2 * i
        pltpu.async_copy(u_hbm.at[src_v.at[j0 + 1]], rows_b, sem_b)
        pltpu.make_async_copy(u_hbm.at[src_v.at[j0]], rows_a, sem_a).wait()
        pltpu.sync_copy(rows_a, acc.at[dst_v.at[j0]], add=True)
        jn = jnp.minimum(j0 + 2, _G - 1)
        pltpu.async_copy(u_hbm.at[src_v.at[jn]], rows_a, sem_a)
        pltpu.make_async_copy(u_hbm.at[src_v.at[j0 + 1]], rows_b, sem_b).wait()
        pltpu.sync_copy(rows_b, acc.at[dst_v.at[j0 + 1]], add=True)
        return carry
    lax.fori_loop(0, _G // 2, pair, 0)
    pltpu.make_async_copy(u_hbm.at[src_v.at[_G - 1]], rows_a, sem_a).wait()


def _make_sc_agg(n_out, n_passes, n_groups, src3d, f):
    @functools.partial(
        pl.kernel,
        mesh=_mesh,
        compiler_params=_params,
        out_type=jax.ShapeDtypeStruct((n_out, _NPAD, f), jnp.float32),
        scratch_types=[
            pltpu.VMEM((_G, 128), jnp.int32),
            pltpu.VMEM((_G, 128), jnp.int32),
            pltpu.VMEM((128, f), jnp.float32),
            pltpu.VMEM((128, f), jnp.float32),
            pltpu.VMEM((_CHK, f), jnp.float32),
            pltpu.VMEM_SHARED((_NPAD, f), jnp.float32),
            pltpu.SemaphoreType.DMA,
            pltpu.SemaphoreType.DMA,
        ],
    )
    def agg(src_hbm, dst_hbm, u_hbm, z_hbm, out_hbm,
            src_v, dst_v, rows_a, rows_b, vbuf, acc, sem_a, sem_b):
        c = lax.axis_index("c")
        s = lax.axis_index("s")
        for p in range(n_passes):
            _zero_acc(z_hbm, vbuf, acc, s)
            plsc.subcore_barrier()
            if src3d:
                gidx = n_passes * c + p
                base = s * (n_groups * _G)
            else:
                gidx = c
                base = c * (_R // 2) + s * (n_groups * _G)
            def group(g, carry):
                row0 = base + g * _G
                if src3d:
                    pltpu.sync_copy(src_hbm.at[gidx, pl.ds(row0, _G)], src_v)
                else:
                    pltpu.sync_copy(src_hbm.at[pl.ds(row0, _G)], src_v)
                pltpu.sync_copy(dst_hbm.at[pl.ds(row0, _G)], dst_v)
                _agg_group(u_hbm, acc, src_v, dst_v, rows_a, rows_b, sem_a, sem_b)
                return carry
            lax.fori_loop(0, n_groups, group, 0)
            plsc.subcore_barrier()
            _drain_acc(acc, vbuf, out_hbm, gidx, s)
            if p + 1 < n_passes:
                plsc.subcore_barrier()
    return agg


_sc_agg1 = _make_sc_agg(2, 1, 7, False, _F)
_sc_agg2 = _make_sc_agg(2, 1, 14, True, 16)


_BR = 1000
_GRID = _N // _BR


def _tc_pre_body(d0, d1, x, dinv_o, u1_o):
    deg = d0[...] + d1[...] + 1.0
    dv = lax.rsqrt(deg)
    dinv_o[...] = dv
    u = dv * x[...]
    u1_o[...] = jnp.concatenate([u, jnp.zeros((_BR, 5), jnp.float32)], axis=1)


def _tc_pre(d0, d1, x):
    return pl.pallas_call(
        _tc_pre_body,
        grid=(_GRID,),
        in_specs=[
            pl.BlockSpec((_BR, 1), lambda i: (i, 0)),
            pl.BlockSpec((_BR, 1), lambda i: (i, 0)),
            pl.BlockSpec((_BR, 3), lambda i: (i, 0)),
        ],
        out_specs=[
            pl.BlockSpec((_BR, 1), lambda i: (i, 0)),
            pl.BlockSpec((_BR, 8), lambda i: (i, 0)),
        ],
        out_shape=[
            jax.ShapeDtypeStruct((_N, 1), jnp.float32),
            jax.ShapeDtypeStruct((_N, 8), jnp.float32),
        ],
    )(d0, d1, x)


def _tc_mid_body(a0, a1, u1, dinv, W1, b1, W2, u2_o):
    s1 = a0[...] + a1[...] + u1[...]
    sn = dinv[...] * s1
    h1 = jnp.maximum(
        jnp.dot(sn[:, :3], W1[...], preferred_element_type=jnp.float32, precision=lax.Precision.HIGHEST) + b1[...],
        0.0)
    t = jnp.dot(h1, W2[...], preferred_element_type=jnp.float32, precision=lax.Precision.HIGHEST)
    u2 = dinv[...] * t
    for g in range(2):
        u2_o[g] = u2[:, 16 * g:16 * g + 16]


def _tc_mid(a0, a1, u1, dinv, W1, b1, W2):
    return pl.pallas_call(
        _tc_mid_body,
        grid=(_GRID,),
        in_specs=[
            pl.BlockSpec((_BR, 8), lambda i: (i, 0)),
            pl.BlockSpec((_BR, 8), lambda i: (i, 0)),
            pl.BlockSpec((_BR, 8), lambda i: (i, 0)),
            pl.BlockSpec((_BR, 1), lambda i: (i, 0)),
            pl.BlockSpec((3, 64), lambda i: (0, 0)),
            pl.BlockSpec((64,), lambda i: (0,)),
            pl.BlockSpec((64, 32), lambda i: (0, 0)),
        ],
        out_specs=pl.BlockSpec((2, _BR, 16), lambda i: (0, i, 0)),
        out_shape=jax.ShapeDtypeStruct((2, _N, 16), jnp.float32),
    )(a0, a1, u1, dinv, W1, b1, W2)


def _tc_head_body(a2, u2, dinv, b2, Wh1, bh1, Wh2, bh2, p_o):
    s2 = a2[...] + u2[...]
    emb = jnp.concatenate([s2[g] for g in range(2)], axis=1)
    emb = dinv[...] * emb + b2[...]
    z = jnp.maximum(
        jnp.dot(emb, Wh1[...], preferred_element_type=jnp.float32, precision=lax.Precision.HIGHEST) + bh1[...],
        0.0)
    p_o[...] = jnp.dot(z, Wh2[...], preferred_element_type=jnp.float32, precision=lax.Precision.HIGHEST) + bh2[...]


def _tc_head(a2, u2, dinv, b2, Wh1, bh1, Wh2, bh2):
    return pl.pallas_call(
        _tc_head_body,
        grid=(_GRID,),
        in_specs=[
            pl.BlockSpec((2, _BR, 16), lambda i: (0, i, 0)),
            pl.BlockSpec((2, _BR, 16), lambda i: (0, i, 0)),
            pl.BlockSpec((_BR, 1), lambda i: (i, 0)),
            pl.BlockSpec((32,), lambda i: (0,)),
            pl.BlockSpec((32, 32), lambda i: (0, 0)),
            pl.BlockSpec((32,), lambda i: (0,)),
            pl.BlockSpec((32, 1), lambda i: (0, 0)),
            pl.BlockSpec((1,), lambda i: (0,)),
        ],
        out_specs=pl.BlockSpec((_BR, 1), lambda i: (i, 0)),
        out_shape=jax.ShapeDtypeStruct((_N, 1), jnp.float32),
    )(a2, u2, dinv, b2, Wh1, bh1, Wh2, bh2)


def kernel(x, edge_index, W1, b1, W2, b2, Wh1, bh1, Wh2, bh2):
    ei = edge_index.astype(jnp.int32)
    padn = _EP - _E
    src_p = jnp.concatenate([ei[0], jnp.zeros((padn,), jnp.int32)])
    dst_p = jnp.concatenate([ei[1], jnp.full((padn,), _N, jnp.int32)])
    src_r = src_p.reshape(_R, 128)
    dst_r = dst_p.reshape(_R, 128)
    src2_r = jnp.stack([src_r + g * _N for g in range(2)])
    z1 = jnp.zeros((_HRPT,), jnp.float32)
    z8 = jnp.zeros((_CHK, _F), jnp.float32)
    z16 = jnp.zeros((_CHK, 16), jnp.float32)

    degs = _sc_degree(dst_r, z1).reshape(2, _NPAD)
    d0 = degs[0, :_N].reshape(_N, 1)
    d1 = degs[1, :_N].reshape(_N, 1)
    dinv, u1 = _tc_pre(d0, d1, x)

    acc1 = _sc_agg1(src_r, dst_r, u1, z8)
    u2 = _tc_mid(acc1[0, :_N], acc1[1, :_N], u1, dinv, W1, b1, W2)

    acc2 = _sc_agg2(src2_r, dst_r, u2.reshape(2 * _N, 16), z16)
    p = _tc_head(acc2[:, :_N, :], u2, dinv, b2, Wh1, bh1, Wh2, bh2)
    return p.reshape(_N)
